# baseline (speedup 1.0000x reference)
import functools

import jax
import jax.numpy as jnp
from jax import lax
from jax.experimental import pallas as pl
from jax.experimental.pallas import tpu as pltpu
from jax.experimental.pallas import tpu_sc as plsc

_B = 16384
_F = 64
_NC = 2
_NS = 16
_NW = _NC * _NS
_BPW = _B // _NW
_G = 16
_IDXROW = 128
_BATCH = 128
_STAGE_ROWS = _B + _NW * 2 * _BATCH
_RING = 4

_UBLOCKS = 7813
_UB_PER_W = 244
_UB_MAX = 249
_UCAP = 32
_IBLOCKS = 782
_IB_MAX = 25
_ICAP = 64
_LCAP = 1024


def _splat(x):
    return jnp.full((_G,), x, jnp.int32)


def _extract_body(nblk_max, cap, blo_fn, bhi_fn):
    def body(ids_hbm, tab_hbm, stage_hbm,
             idsv, listb, listid, memb, bb, rowbuf,
             bidx0, bidx1, cnt_s, nblist_s, nnb_s, bsem, ssem):
        wid = lax.axis_index("s") * _NC + lax.axis_index("c")
        blo = blo_fn(wid)
        bhi = bhi_fn(wid)
        lanes = lax.iota(jnp.int32, _G)

        pltpu.sync_copy(ids_hbm, idsv)

        def filt(t, off):
            vec = idsv[t // 8, pl.ds((t % 8) * _G, _G)]
            blk = vec >> 7
            m = (blk >= blo) & (blk < bhi)
            bvec = t * _G + lanes
            plsc.store_compressed(listb.at[pl.ds(off, _G)], bvec, mask=m)
            plsc.store_compressed(listid.at[pl.ds(off, _G)], vec, mask=m)
            return off + plsc.all_reduce_population_count(m)[0]

        count = lax.fori_loop(0, (_B // _G), filt, jnp.int32(0),
                              unroll=False)

        def zero(k, c):
            cnt_s[k] = jnp.int32(0)
            return c
        lax.fori_loop(0, nblk_max, zero, 0, unroll=False)
        nnb_s[0] = jnp.int32(0)

        def bucket(t, c):
            id16 = listid[pl.ds(t * _G, _G)]
            for i in range(_G):
                @pl.when(t * _G + i < count)
                def _():
                    k = (id16[i] >> 7) - blo
                    n = cnt_s[k]
                    cnt_s[k] = n + 1

                    @pl.when(n == 0)
                    def _():
                        j = nnb_s[0]
                        nblist_s[j] = k
                        nnb_s[0] = j + 1

                    plsc.store_scatter(
                        memb, [_splat(k * cap + n)], _splat(t * _G + i),
                        mask=lanes == 0)
            return c
        lax.fori_loop(0, (count + _G - 1) // _G, bucket, 0, unroll=False)
        nnb = nnb_s[0]

        def fetch(j):
            k = nblist_s[j]
            c0 = pl.multiple_of((blo + k) * 128, 128)
            s = lax.rem(j, _RING)
            return pltpu.make_async_copy(
                tab_hbm.at[:, pl.ds(c0, 128)],
                bb.at[pl.ds(s * _F, _F), :], bsem.at[s])

        for j0 in range(_RING):
            @pl.when(j0 < nnb)
            def _():
                fetch(j0).start()

        def park_bidx(h, ref):
            for q in range(_BATCH // _G):
                ref[pl.ds(q * _G, _G)] = (_B + wid * 2 * _BATCH
                                          + h * _BATCH + q * _G + lanes)

        park_bidx(0, bidx0)
        park_bidx(1, bidx1)

        def _half(h):
            return (rowbuf.at[pl.ds(h * _BATCH, _BATCH), :],
                    bidx0 if h == 0 else bidx1)

        def _flush(h, nbat2):
            buf, bidx = _half(h)
            pltpu.async_copy(buf, stage_hbm.at[bidx], ssem)

            @pl.when(nbat2 >= 1)
            def _():
                obuf, _o = _half(1 - h)
                pltpu.make_async_copy(stage_hbm.at[pl.ds(0, _BATCH), :],
                                      obuf, ssem).wait()

            park_bidx(1 - h, bidx0 if h == 1 else bidx1)

        def sweep(j, carry):
            slot, nbat = carry
            k = nblist_s[j]
            n = cnt_s[k]
            fetch(j).wait()
            rbase = lax.rem(j, _RING) * _F

            def member(m, carry2):
                slot2, nbat2 = carry2
                h = lax.rem(nbat2, 2)
                ptr = plsc.load_gather(memb, [_splat(k * cap + m)])[0]
                b = plsc.load_gather(listb, [_splat(ptr)])[0]
                mid = plsc.load_gather(listid, [_splat(ptr)])[0]
                c = mid & 127
                row = h * _BATCH + slot2
                for k4 in range(_F // _G):
                    q = plsc.load_gather(bb, [rbase + k4 * _G + lanes,
                                              _splat(c)])
                    rowbuf[row, pl.ds(k4 * _G, _G)] = q

                @pl.when(h == 0)
                def _():
                    plsc.store_scatter(bidx0, [_splat(slot2)], _splat(b),
                                       mask=lanes == 0)

                @pl.when(h == 1)
                def _():
                    plsc.store_scatter(bidx1, [_splat(slot2)], _splat(b),
                                       mask=lanes == 0)

                full = slot2 == (_BATCH - 1)

                @pl.when(full & (h == 0))
                def _():
                    _flush(0, nbat2)

                @pl.when(full & (h == 1))
                def _():
                    _flush(1, nbat2)

                return (jnp.where(full, 0, slot2 + 1),
                        nbat2 + jnp.where(full, jnp.int32(1), jnp.int32(0)))

            carry_out = lax.fori_loop(0, n, member, (slot, nbat),
                                      unroll=False)

            @pl.when(j + _RING < nnb)
            def _():
                fetch(j + _RING).start()

            return carry_out

        slot, nbat = lax.fori_loop(0, nnb, sweep,
                                   (jnp.int32(0), jnp.int32(0)),
                                   unroll=False)

        hf = lax.rem(nbat, 2)

        @pl.when(hf == 0)
        def _():
            buf, bidx = _half(0)
            pltpu.async_copy(buf, stage_hbm.at[bidx], ssem)

        @pl.when(hf == 1)
        def _():
            buf, bidx = _half(1)
            pltpu.async_copy(buf, stage_hbm.at[bidx], ssem)

        def _drain(i, c):
            pltpu.make_async_copy(
                stage_hbm.at[pl.ds(0, _BATCH), :],
                rowbuf.at[pl.ds(0, _BATCH), :], ssem).wait()
            return c

        lax.fori_loop(0, jnp.minimum(nbat, 1) + 1, _drain, 0, unroll=False)

    return body


def _dot_body(ustage_hbm, istage_hbm, out_hbm, ublk, iblk, out_v, sem):
    wid = lax.axis_index("s") * _NC + lax.axis_index("c")
    lanes = lax.iota(jnp.int32, _G)
    half_rows = _BPW // 2

    for h in range(2):
        base = wid * _BPW + h * half_rows
        pltpu.sync_copy(ustage_hbm.at[pl.ds(base, half_rows)], ublk)
        pltpu.sync_copy(istage_hbm.at[pl.ds(base, half_rows)], iblk)

        def group(g, c):
            res = jnp.zeros((_G,), jnp.float32)
            for i in range(_G):
                r = g * _G + i
                acc = ublk[r, pl.ds(0, _G)] * iblk[r, pl.ds(0, _G)]
                for k in range(1, _F // _G):
                    acc = acc + (ublk[r, pl.ds(k * _G, _G)]
                                 * iblk[r, pl.ds(k * _G, _G)])
                res = jnp.where(lanes == i, jnp.sum(acc), res)
            out_v[pl.ds(h * half_rows + g * _G, _G)] = res
            return c

        lax.fori_loop(0, half_rows // _G, group, 0, unroll=False)

    pltpu.sync_copy(out_v, out_hbm.at[pl.ds(wid * _BPW, _BPW)])


def _mesh():
    return plsc.VectorSubcoreMesh(core_axis_name="c", subcore_axis_name="s")


def _extract_kernel(nblk_max, cap, blo_fn, bhi_fn):
    return functools.partial(
        pl.kernel,
        out_type=jax.ShapeDtypeStruct((_STAGE_ROWS, 128), jnp.float32),
        mesh=_mesh(),
        scratch_types=[
            pltpu.VMEM((_B // _IDXROW, _IDXROW), jnp.int32),
            pltpu.VMEM((_LCAP,), jnp.int32),
            pltpu.VMEM((_LCAP,), jnp.int32),
            pltpu.VMEM((nblk_max * cap,), jnp.int32),
            pltpu.VMEM((_RING * _F, 128), jnp.float32),
            pltpu.VMEM((2 * _BATCH, 128), jnp.float32),
            pltpu.VMEM((_BATCH,), jnp.int32),
            pltpu.VMEM((_BATCH,), jnp.int32),
            pltpu.SMEM((nblk_max,), jnp.int32),
            pltpu.SMEM((nblk_max,), jnp.int32),
            pltpu.SMEM((1,), jnp.int32),
            pltpu.SemaphoreType.DMA((_RING,)),
            pltpu.SemaphoreType.DMA,
        ],
        compiler_params=pltpu.CompilerParams(
            needs_layout_passes=False, use_tc_tiling_on_sc=True),
    )(_extract_body(nblk_max, cap, blo_fn, bhi_fn))


@jax.jit
def _cfmodel_g(uid, iid, ut_t, it_t):
    u_extract = _extract_kernel(
        _UB_MAX, _UCAP,
        lambda w: w * _UB_PER_W,
        lambda w: jnp.where(w == _NW - 1, _UBLOCKS, (w + 1) * _UB_PER_W))
    i_extract = _extract_kernel(
        _IB_MAX, _ICAP,
        lambda w: 24 * w + jnp.minimum(w, 14),
        lambda w: 24 * (w + 1) + jnp.minimum(w + 1, 14))
    ustage = u_extract(uid, ut_t)
    istage = i_extract(iid, it_t)

    dot = functools.partial(
        pl.kernel,
        out_type=jax.ShapeDtypeStruct((_B,), jnp.float32),
        mesh=_mesh(),
        scratch_types=[
            pltpu.VMEM((_BPW // 2, 128), jnp.float32),
            pltpu.VMEM((_BPW // 2, 128), jnp.float32),
            pltpu.VMEM((_BPW,), jnp.float32),
            pltpu.SemaphoreType.DMA,
        ],
        compiler_params=pltpu.CompilerParams(
            needs_layout_passes=False, use_tc_tiling_on_sc=True),
    )(_dot_body)
    return dot(ustage, istage)


def kernel(input_user_id, input_item_id, user_table, item_table):
    uid = input_user_id.reshape(_B // _IDXROW, _IDXROW).astype(jnp.int32)
    iid = input_item_id.reshape(_B // _IDXROW, _IDXROW).astype(jnp.int32)
    out = _cfmodel_g(uid, iid, user_table.T, item_table.T)
    return out.reshape(_B, 1)


# speedup vs baseline: 1.0089x; 1.0089x over previous
import functools

import jax
import jax.numpy as jnp
from jax import lax
from jax.experimental import pallas as pl
from jax.experimental.pallas import tpu as pltpu
from jax.experimental.pallas import tpu_sc as plsc

_B = 16384
_F = 64
_NC = 2
_NS = 16
_NW = _NC * _NS
_BPW = _B // _NW
_G = 16
_IDXROW = 128
_BATCH = 64
_STAGE_ROWS = _B + _NW * _BATCH
_RING = 4

_UBLOCKS = 7813
_UB_PER_W = 244
_UB_MAX = 249
_UCAP = 32
_IBLOCKS = 782
_IB_MAX = 25
_ICAP = 64
_LCAP = 1024


def _splat(x):
    return jnp.full((_G,), x, jnp.int32)


def _extract_body(nblk_max, cap, blo_fn, bhi_fn):
    def body(ids_hbm, tab_hbm, stage_hbm,
             idsv, listb, listid, memb, bb, rowbuf, bidx_v,
             cnt_s, nblist_s, nnb_s, bsem):
        wid = lax.axis_index("s") * _NC + lax.axis_index("c")
        blo = blo_fn(wid)
        bhi = bhi_fn(wid)
        lanes = lax.iota(jnp.int32, _G)

        pltpu.sync_copy(ids_hbm, idsv)

        def filt(t, off):
            vec = idsv[t // 8, pl.ds((t % 8) * _G, _G)]
            blk = vec >> 7
            m = (blk >= blo) & (blk < bhi)
            bvec = t * _G + lanes
            plsc.store_compressed(listb.at[pl.ds(off, _G)], bvec, mask=m)
            plsc.store_compressed(listid.at[pl.ds(off, _G)], vec, mask=m)
            return off + plsc.all_reduce_population_count(m)[0]

        count = lax.fori_loop(0, (_B // _G), filt, jnp.int32(0),
                              unroll=False)

        def zero(k, c):
            cnt_s[k] = jnp.int32(0)
            return c
        lax.fori_loop(0, nblk_max, zero, 0, unroll=False)
        nnb_s[0] = jnp.int32(0)

        def bucket(t, c):
            id16 = listid[pl.ds(t * _G, _G)]
            for i in range(_G):
                @pl.when(t * _G + i < count)
                def _():
                    k = (id16[i] >> 7) - blo
                    n = cnt_s[k]
                    cnt_s[k] = n + 1

                    @pl.when(n == 0)
                    def _():
                        j = nnb_s[0]
                        nblist_s[j] = k
                        nnb_s[0] = j + 1

                    plsc.store_scatter(
                        memb, [_splat(k * cap + n)], _splat(t * _G + i),
                        mask=lanes == 0)
            return c
        lax.fori_loop(0, (count + _G - 1) // _G, bucket, 0, unroll=False)
        nnb = nnb_s[0]

        def fetch(j):
            k = nblist_s[j]
            c0 = pl.multiple_of((blo + k) * 128, 128)
            s = lax.rem(j, _RING)
            return pltpu.make_async_copy(
                tab_hbm.at[:, pl.ds(c0, 128)],
                bb.at[pl.ds(s * _F, _F), :], bsem.at[s])

        for j0 in range(_RING):
            @pl.when(j0 < nnb)
            def _():
                fetch(j0).start()

        def park_bidx():
            for q in range(_BATCH // _G):
                bidx_v[pl.ds(q * _G, _G)] = (_B + wid * _BATCH + q * _G
                                             + lanes)

        park_bidx()

        def sweep(j, carry):
            slot = carry
            k = nblist_s[j]
            n = cnt_s[k]
            fetch(j).wait()
            rbase = lax.rem(j, _RING) * _F

            def member(m, slot2):
                ptr = plsc.load_gather(memb, [_splat(k * cap + m)])[0]
                b = plsc.load_gather(listb, [_splat(ptr)])[0]
                mid = plsc.load_gather(listid, [_splat(ptr)])[0]
                c = mid & 127
                for k4 in range(_F // _G):
                    q = plsc.load_gather(bb, [rbase + k4 * _G + lanes,
                                              _splat(c)])
                    rowbuf[slot2, pl.ds(k4 * _G, _G)] = q
                plsc.store_scatter(bidx_v, [_splat(slot2)], _splat(b),
                                   mask=lanes == 0)
                full = slot2 == (_BATCH - 1)

                @pl.when(full)
                def _():
                    pltpu.sync_copy(rowbuf, stage_hbm.at[bidx_v])
                    park_bidx()

                return jnp.where(full, 0, slot2 + 1)

            slot_out = lax.fori_loop(0, n, member, slot, unroll=False)

            @pl.when(j + _RING < nnb)
            def _():
                fetch(j + _RING).start()

            return slot_out

        lax.fori_loop(0, nnb, sweep, jnp.int32(0), unroll=False)

        pltpu.sync_copy(rowbuf, stage_hbm.at[bidx_v])

    return body


def _dot_body(ustage_hbm, istage_hbm, out_hbm, ublk, iblk, out_v, sem):
    wid = lax.axis_index("s") * _NC + lax.axis_index("c")
    lanes = lax.iota(jnp.int32, _G)
    half_rows = _BPW // 2

    for h in range(2):
        base = wid * _BPW + h * half_rows
        pltpu.sync_copy(ustage_hbm.at[pl.ds(base, half_rows)], ublk)
        pltpu.sync_copy(istage_hbm.at[pl.ds(base, half_rows)], iblk)

        def group(g, c):
            res = jnp.zeros((_G,), jnp.float32)
            for i in range(_G):
                r = g * _G + i
                acc = ublk[r, pl.ds(0, _G)] * iblk[r, pl.ds(0, _G)]
                for k in range(1, _F // _G):
                    acc = acc + (ublk[r, pl.ds(k * _G, _G)]
                                 * iblk[r, pl.ds(k * _G, _G)])
                res = jnp.where(lanes == i, jnp.sum(acc), res)
            out_v[pl.ds(h * half_rows + g * _G, _G)] = res
            return c

        lax.fori_loop(0, half_rows // _G, group, 0, unroll=False)

    pltpu.sync_copy(out_v, out_hbm.at[pl.ds(wid * _BPW, _BPW)])


def _mesh():
    return plsc.VectorSubcoreMesh(core_axis_name="c", subcore_axis_name="s")


def _extract_kernel(nblk_max, cap, blo_fn, bhi_fn):
    return functools.partial(
        pl.kernel,
        out_type=jax.ShapeDtypeStruct((_STAGE_ROWS, 128), jnp.float32),
        mesh=_mesh(),
        scratch_types=[
            pltpu.VMEM((_B // _IDXROW, _IDXROW), jnp.int32),
            pltpu.VMEM((_LCAP,), jnp.int32),
            pltpu.VMEM((_LCAP,), jnp.int32),
            pltpu.VMEM((nblk_max * cap,), jnp.int32),
            pltpu.VMEM((_RING * _F, 128), jnp.float32),
            pltpu.VMEM((_BATCH, 128), jnp.float32),
            pltpu.VMEM((_BATCH,), jnp.int32),
            pltpu.SMEM((nblk_max,), jnp.int32),
            pltpu.SMEM((nblk_max,), jnp.int32),
            pltpu.SMEM((1,), jnp.int32),
            pltpu.SemaphoreType.DMA((_RING,)),
        ],
        compiler_params=pltpu.CompilerParams(
            needs_layout_passes=False, use_tc_tiling_on_sc=True),
    )(_extract_body(nblk_max, cap, blo_fn, bhi_fn))


@jax.jit
def _cfmodel_g(uid, iid, ut_t, it_t):
    u_extract = _extract_kernel(
        _UB_MAX, _UCAP,
        lambda w: w * _UB_PER_W,
        lambda w: jnp.where(w == _NW - 1, _UBLOCKS, (w + 1) * _UB_PER_W))
    i_extract = _extract_kernel(
        _IB_MAX, _ICAP,
        lambda w: 24 * w + jnp.minimum(w, 14),
        lambda w: 24 * (w + 1) + jnp.minimum(w + 1, 14))
    ustage = u_extract(uid, ut_t)
    istage = i_extract(iid, it_t)

    dot = functools.partial(
        pl.kernel,
        out_type=jax.ShapeDtypeStruct((_B,), jnp.float32),
        mesh=_mesh(),
        scratch_types=[
            pltpu.VMEM((_BPW // 2, 128), jnp.float32),
            pltpu.VMEM((_BPW // 2, 128), jnp.float32),
            pltpu.VMEM((_BPW,), jnp.float32),
            pltpu.SemaphoreType.DMA,
        ],
        compiler_params=pltpu.CompilerParams(
            needs_layout_passes=False, use_tc_tiling_on_sc=True),
    )(_dot_body)
    return dot(ustage, istage)


def kernel(input_user_id, input_item_id, user_table, item_table):
    uid = input_user_id.reshape(_B // _IDXROW, _IDXROW).astype(jnp.int32)
    iid = input_item_id.reshape(_B // _IDXROW, _IDXROW).astype(jnp.int32)
    out = _cfmodel_g(uid, iid, user_table.T, item_table.T)
    return out.reshape(_B, 1)
